# baseline (speedup 1.0000x reference)
import functools

import jax
import jax.numpy as jnp
from jax import lax
from jax.experimental import pallas as pl
from jax.experimental.pallas import tpu as pltpu
from jax.experimental.pallas import tpu_sc as plsc

_L = 16


@functools.cache
def _build(batch, feat_dim, num_classes):
    info = plsc.get_sparse_core_info()
    nc, ns = info.num_cores, info.num_subcores
    nw = nc * ns
    b_per_w = batch // nw
    n_chunk = 128
    chunks = b_per_w // n_chunk
    groups = feat_dim // _L
    scale = 0.5 / batch

    mesh = plsc.VectorSubcoreMesh(core_axis_name="c", subcore_axis_name="s")

    @functools.partial(
        pl.kernel,
        out_type=jax.ShapeDtypeStruct((nw, _L), jnp.float32),
        mesh=mesh,
        compiler_params=pltpu.CompilerParams(use_tc_tiling_on_sc=True),
        scratch_types=[
            pltpu.VMEM((b_per_w,), jnp.int32),
            pltpu.VMEM((b_per_w,), jnp.int32),
            pltpu.VMEM((b_per_w,), jnp.int32),
            pltpu.VMEM((b_per_w, 2 * feat_dim), jnp.float32),
            pltpu.VMEM((b_per_w // 2, 2 * feat_dim), jnp.float32),
            pltpu.VMEM((_L,), jnp.float32),
            pltpu.SemaphoreType.DMA,
            pltpu.SemaphoreType.DMA,
        ],
    )
    def k(feat_hbm, lab_hbm, cent_hbm, out_hbm, lab_v, idx_v, par_v, rows_v,
          feat_v, acc_v, gsem, fsem):
        wid = lax.axis_index("s") * nc + lax.axis_index("c")
        base = wid * b_per_w

        fcopy = pltpu.async_copy(
            feat_hbm.at[pl.ds(wid * (b_per_w // 2), b_per_w // 2)], feat_v, fsem)
        pltpu.sync_copy(lab_hbm.at[pl.ds(base, b_per_w)], lab_v)

        def split(i, _):
            l = lab_v[pl.ds(i * _L, _L)]
            idx_v[pl.ds(i * _L, _L)] = lax.shift_right_logical(l, 1)
            par_v[pl.ds(i * _L, _L)] = lax.bitwise_and(l, 1)
            return 0

        lax.fori_loop(0, b_per_w // _L, split, 0)

        copies = [
            pltpu.async_copy(
                cent_hbm.at[idx_v.at[pl.ds(j * n_chunk, n_chunk)]],
                rows_v.at[pl.ds(j * n_chunk, n_chunk)],
                gsem,
            )
            for j in range(chunks)
        ]
        for c in copies:
            c.wait()
        fcopy.wait()

        def body(ib, acc):
            pvec = par_v[pl.ds(ib * _L, _L)] * feat_dim
            for j in range(_L):
                i = ib * _L + j
                off = pvec[j]
                for g in range(groups):
                    f = feat_v[ib * (_L // 2) + j // 2,
                               pl.ds((j % 2) * feat_dim + g * _L, _L)]
                    c = rows_v[i, pl.ds(off + g * _L, _L)]
                    d = f - c
                    acc = acc + d * d
            return acc

        acc = lax.fori_loop(0, b_per_w // _L, body, jnp.zeros((_L,), jnp.float32))
        acc_v[...] = acc * scale
        pltpu.sync_copy(acc_v, out_hbm.at[wid])

    return k


def kernel(features, labels, centers):
    batch, feat_dim = features.shape
    num_classes = centers.shape[0]
    k = _build(batch, feat_dim, num_classes)
    cent2 = centers.reshape(num_classes // 2, 2 * feat_dim)
    feat2 = features.reshape(batch // 2, 2 * feat_dim)
    partials = k(feat2, labels.astype(jnp.int32), cent2)
    return jnp.sum(partials)


# speedup vs baseline: 2.2941x; 2.2941x over previous
import functools

import jax
import jax.numpy as jnp
from jax import lax
from jax.experimental import pallas as pl
from jax.experimental.pallas import tpu as pltpu
from jax.experimental.pallas import tpu_sc as plsc

_L = 16
_UNROLL = 4


@functools.cache
def _build(batch, feat_dim, num_classes):
    info = plsc.get_sparse_core_info()
    nc, ns = info.num_cores, info.num_subcores
    nw = nc * ns
    rows_per_w = feat_dim // nw
    f_chunk = 8192
    n_fchunk = batch // f_chunk
    scale = 0.5 / batch

    mesh = plsc.VectorSubcoreMesh(core_axis_name="c", subcore_axis_name="s")

    @functools.partial(
        pl.kernel,
        out_type=jax.ShapeDtypeStruct((nw, _L), jnp.float32),
        mesh=mesh,
        compiler_params=pltpu.CompilerParams(
            use_tc_tiling_on_sc=True, needs_layout_passes=False),
        scratch_types=[
            pltpu.VMEM((num_classes,), jnp.float32),
            pltpu.VMEM((batch,), jnp.int32),
            pltpu.VMEM((f_chunk,), jnp.float32),
            pltpu.VMEM((_L,), jnp.float32),
            pltpu.SemaphoreType.DMA,
            pltpu.SemaphoreType.DMA,
        ],
    )
    def k(featT_hbm, lab_hbm, centT_hbm, out_hbm, row_v, lab_v, feat_v,
          acc_v, rsem, fsem):
        wid = lax.axis_index("s") * nc + lax.axis_index("c")

        rcopy = pltpu.async_copy(centT_hbm.at[wid * rows_per_w], row_v, rsem)
        pltpu.sync_copy(lab_hbm, lab_v)

        acc = jnp.zeros((_L,), jnp.float32)
        for r in range(rows_per_w):
            f = wid * rows_per_w + r
            rcopy.wait()
            for h in range(n_fchunk):
                fcopy = pltpu.async_copy(
                    featT_hbm.at[f, pl.ds(h * f_chunk, f_chunk)], feat_v, fsem)
                if h == n_fchunk - 1 and r < rows_per_w - 1:
                    pass
                fcopy.wait()

                def body(it, acc):
                    base = it * (_L * _UNROLL)
                    for u in range(_UNROLL):
                        o = base + u * _L
                        idx = lab_v[pl.ds(h * f_chunk + o, _L)]
                        c = plsc.load_gather(row_v, [idx])
                        fv = feat_v[pl.ds(o, _L)]
                        d = fv - c
                        acc = acc + d * d
                    return acc

                acc = lax.fori_loop(0, f_chunk // (_L * _UNROLL), body, acc)
            if r < rows_per_w - 1:
                rcopy = pltpu.async_copy(
                    centT_hbm.at[wid * rows_per_w + r + 1], row_v, rsem)

        acc_v[...] = acc * scale
        pltpu.sync_copy(acc_v, out_hbm.at[wid])

    return k


def kernel(features, labels, centers):
    batch, feat_dim = features.shape
    num_classes = centers.shape[0]
    k = _build(batch, feat_dim, num_classes)
    partials = k(features.T, labels.astype(jnp.int32), centers.T)
    return jnp.sum(partials)


# speedup vs baseline: 2.4224x; 1.0559x over previous
import functools

import jax
import jax.numpy as jnp
from jax import lax
from jax.experimental import pallas as pl
from jax.experimental.pallas import tpu as pltpu
from jax.experimental.pallas import tpu_sc as plsc

_L = 16
_UNROLL = 8


@functools.cache
def _build(batch, feat_dim, num_classes):
    info = plsc.get_sparse_core_info()
    nc, ns = info.num_cores, info.num_subcores
    nw = nc * ns
    rows_per_w = feat_dim // nw
    f_chunk = 4096
    n_fchunk = batch // f_chunk
    scale = 0.5 / batch

    mesh = plsc.VectorSubcoreMesh(core_axis_name="c", subcore_axis_name="s")

    @functools.partial(
        pl.kernel,
        out_type=jax.ShapeDtypeStruct((nw, _L), jnp.float32),
        mesh=mesh,
        compiler_params=pltpu.CompilerParams(
            use_tc_tiling_on_sc=True, needs_layout_passes=False),
        scratch_types=[
            pltpu.VMEM((num_classes,), jnp.float32),
            pltpu.VMEM((batch,), jnp.int32),
            pltpu.VMEM((f_chunk,), jnp.float32),
            pltpu.VMEM((f_chunk,), jnp.float32),
            pltpu.VMEM((_L,), jnp.float32),
            pltpu.SemaphoreType.DMA,
            pltpu.SemaphoreType.DMA,
            pltpu.SemaphoreType.DMA,
        ],
    )
    def k(featT_hbm, lab_hbm, centT_hbm, out_hbm, row_v, lab_v, feat_v0,
          feat_v1, acc_v, rsem, fsem0, fsem1):
        wid = lax.axis_index("s") * nc + lax.axis_index("c")

        rcopy = pltpu.async_copy(centT_hbm.at[wid * rows_per_w], row_v, rsem)
        pltpu.sync_copy(lab_hbm, lab_v)

        fbufs = (feat_v0, feat_v1)
        fsems = (fsem0, fsem1)

        acc = jnp.zeros((_L,), jnp.float32)
        for r in range(rows_per_w):
            f = wid * rows_per_w + r
            fcopies = [None] * n_fchunk
            fcopies[0] = pltpu.async_copy(
                featT_hbm.at[f, pl.ds(0, f_chunk)], fbufs[0], fsems[0])
            rcopy.wait()
            for h in range(n_fchunk):
                if h + 1 < n_fchunk:
                    b = (h + 1) % 2
                    fcopies[h + 1] = pltpu.async_copy(
                        featT_hbm.at[f, pl.ds((h + 1) * f_chunk, f_chunk)],
                        fbufs[b], fsems[b])
                fcopies[h].wait()
                feat_v = fbufs[h % 2]

                def body(it, acc):
                    base = it * (_L * _UNROLL)
                    for u in range(_UNROLL):
                        o = base + u * _L
                        idx = lab_v[pl.ds(h * f_chunk + o, _L)]
                        c = plsc.load_gather(row_v, [idx])
                        fv = feat_v[pl.ds(o, _L)]
                        d = fv - c
                        acc = acc + d * d
                    return acc

                acc = lax.fori_loop(0, f_chunk // (_L * _UNROLL), body, acc)
            if r < rows_per_w - 1:
                rcopy = pltpu.async_copy(
                    centT_hbm.at[wid * rows_per_w + r + 1], row_v, rsem)

        acc_v[...] = acc * scale
        pltpu.sync_copy(acc_v, out_hbm.at[wid])

    return k


def kernel(features, labels, centers):
    batch, feat_dim = features.shape
    num_classes = centers.shape[0]
    k = _build(batch, feat_dim, num_classes)
    partials = k(features.T, labels.astype(jnp.int32), centers.T)
    return jnp.sum(partials)
